# Initial kernel scaffold; baseline (speedup 1.0000x reference)
import jax
import jax.numpy as jnp
from jax.experimental import pallas as pl


def kernel(feats, embedding):
    raise NotImplementedError("write your pallas kernel here")


# speedup vs baseline: 1.2663x; 1.2663x over previous
import functools

import jax
import jax.numpy as jnp
from jax import lax
from jax.experimental import pallas as pl
from jax.experimental.pallas import tpu as pltpu
from jax.experimental.pallas import tpu_sc as plsc

VOCAB = 4096
DIM = 3
NTOK = 65536
TBLK = 512
NC, NS, LANES = 2, 16, 16
NW = NC * NS
TOK_W = NTOK // NW
GROUPS = TOK_W // LANES


def _tc_argmin_body(f_ref, et_ref, f2_ref, e2_ref, idx_ref):
    f = f_ref[...]
    cv = jax.lax.dot_general(
        f, et_ref[...], (((1,), (0,)), ((), ())),
        preferred_element_type=jnp.float32)
    dist = (f2_ref[...] - cv * 2.0) + e2_ref[...]
    m = jnp.min(dist, axis=1, keepdims=True)
    lane = jax.lax.broadcasted_iota(jnp.int32, dist.shape, 1)
    sel = jnp.where(dist <= m, lane, jnp.int32(VOCAB))
    idx_ref[...] = jnp.min(sel, axis=1)


def _tc_argmin(flat_q, et, f2, e2):
    return pl.pallas_call(
        _tc_argmin_body,
        grid=(NTOK // TBLK,),
        in_specs=[
            pl.BlockSpec((TBLK, DIM), lambda i: (i, 0)),
            pl.BlockSpec((DIM, VOCAB), lambda i: (0, 0)),
            pl.BlockSpec((TBLK, 1), lambda i: (i, 0)),
            pl.BlockSpec((1, VOCAB), lambda i: (0, 0)),
        ],
        out_specs=pl.BlockSpec((TBLK,), lambda i: (i,)),
        out_shape=jax.ShapeDtypeStruct((NTOK,), jnp.int32),
    )(flat_q, et, f2, e2)


def _sc_lookup_body(emb_hbm, idx_hbm, feats_hbm, out_hbm, part_hbm,
                    emb_v, idx_v, f_v, out_v, acc_v):
    wid = lax.axis_index("s") * NC + lax.axis_index("c")
    base = wid * TOK_W
    base3 = base * DIM
    pltpu.sync_copy(emb_hbm, emb_v)
    pltpu.sync_copy(idx_hbm.at[pl.ds(base, TOK_W)], idx_v)
    pltpu.sync_copy(feats_hbm.at[pl.ds(base3, TOK_W * DIM)], f_v)

    iota = lax.iota(jnp.int32, LANES)
    iota3 = iota * 3

    def body(i, acc):
        iv = idx_v[pl.ds(i * LANES, LANES)]
        iv3 = iv * 3
        for c in range(DIM):
            widx = iota3 + (i * (LANES * DIM) + c)
            fv = plsc.load_gather(f_v, [widx])
            ev = plsc.load_gather(emb_v, [iv3 + c])
            d = ev - fv
            plsc.store_scatter(out_v, [widx], fv + d)
            acc = acc + d * d
        return acc

    acc = lax.fori_loop(0, GROUPS, body, jnp.zeros((LANES,), jnp.float32))
    acc_v[...] = acc
    pltpu.sync_copy(out_v, out_hbm.at[pl.ds(base3, TOK_W * DIM)])
    pltpu.sync_copy(acc_v, part_hbm.at[wid])


@functools.cache
def _sc_lookup():
    return pl.kernel(
        _sc_lookup_body,
        out_type=[
            jax.ShapeDtypeStruct((NTOK * DIM,), jnp.float32),
            jax.ShapeDtypeStruct((NW, LANES), jnp.float32),
        ],
        mesh=plsc.VectorSubcoreMesh(core_axis_name="c", subcore_axis_name="s",
                                    num_cores=NC, num_subcores=NS),
        compiler_params=pltpu.CompilerParams(needs_layout_passes=False),
        scratch_types=[
            pltpu.VMEM((VOCAB * DIM,), jnp.float32),
            pltpu.VMEM((TOK_W,), jnp.int32),
            pltpu.VMEM((TOK_W * DIM,), jnp.float32),
            pltpu.VMEM((TOK_W * DIM,), jnp.float32),
            pltpu.VMEM((LANES,), jnp.float32),
        ],
    )


def kernel(feats, embedding):
    B, L, D = feats.shape
    flat = feats.reshape(NTOK, DIM)
    flat_q = flat.astype(jnp.bfloat16).astype(jnp.float32)
    f2 = jnp.sum(flat ** 2, axis=1)[:, None]
    e2 = jnp.sum(embedding ** 2, axis=1)[None, :]
    idx = _tc_argmin(flat_q, embedding.T, f2, e2)
    qst_flat, partials = _sc_lookup()(
        embedding.reshape(-1), idx, flat.reshape(-1))
    quant_st = qst_flat.reshape(B, L, D)
    loss = jnp.sum(partials) * jnp.float32(1.25 / (NTOK * DIM))
    return (quant_st, idx.reshape(B, L), loss)


# speedup vs baseline: 1.4692x; 1.1602x over previous
import functools

import jax
import jax.numpy as jnp
from jax import lax
from jax.experimental import pallas as pl
from jax.experimental.pallas import tpu as pltpu
from jax.experimental.pallas import tpu_sc as plsc

VOCAB = 4096
DIM = 3
NTOK = 65536
TBLK = 1024
NC, NS, LANES = 2, 16, 16
NW = NC * NS
TOK_W = NTOK // NW
GROUPS = TOK_W // LANES


def _tc_argmin_body(f_ref, et_ref, f2_ref, e2_ref, idx_ref):
    f = f_ref[...]
    cv = jax.lax.dot_general(
        f, et_ref[...], (((1,), (0,)), ((), ())),
        preferred_element_type=jnp.float32)
    dist = (f2_ref[...] - cv * 2.0) + e2_ref[...]
    m = jnp.min(dist, axis=1, keepdims=True)
    lane = jax.lax.broadcasted_iota(jnp.int32, (1, VOCAB), 1).astype(jnp.float32)
    sel = jnp.where(dist <= m, lane, jnp.float32(VOCAB))
    idx_ref[...] = jnp.min(sel, axis=1).astype(jnp.int32)


def _tc_argmin(flat_q, et, f2, e2):
    return pl.pallas_call(
        _tc_argmin_body,
        grid=(NTOK // TBLK,),
        in_specs=[
            pl.BlockSpec((TBLK, DIM), lambda i: (i, 0)),
            pl.BlockSpec((DIM, VOCAB), lambda i: (0, 0)),
            pl.BlockSpec((TBLK, 1), lambda i: (i, 0)),
            pl.BlockSpec((1, VOCAB), lambda i: (0, 0)),
        ],
        out_specs=pl.BlockSpec((TBLK,), lambda i: (i,)),
        out_shape=jax.ShapeDtypeStruct((NTOK,), jnp.int32),
    )(flat_q, et, f2, e2)


def _sc_lookup_body(emb_hbm, idx_hbm, feats_hbm, out_hbm, part_hbm,
                    emb_v, idx_v, f_v, out_v, acc_v):
    wid = lax.axis_index("s") * NC + lax.axis_index("c")
    base = wid * TOK_W
    base3 = base * DIM
    pltpu.sync_copy(emb_hbm, emb_v)
    pltpu.sync_copy(idx_hbm.at[pl.ds(base, TOK_W)], idx_v)
    pltpu.sync_copy(feats_hbm.at[pl.ds(base3, TOK_W * DIM)], f_v)

    iota = lax.iota(jnp.int32, LANES)
    iota3 = iota * 3

    def body(i, acc):
        iv = idx_v[pl.ds(i * LANES, LANES)]
        iv3 = iv * 3
        for c in range(DIM):
            widx = iota3 + (i * (LANES * DIM) + c)
            fv = plsc.load_gather(f_v, [widx])
            ev = plsc.load_gather(emb_v, [iv3 + c])
            d = ev - fv
            plsc.store_scatter(out_v, [widx], fv + d)
            acc = acc + d * d
        return acc

    acc = lax.fori_loop(0, GROUPS, body, jnp.zeros((LANES,), jnp.float32))
    acc_v[...] = acc
    pltpu.sync_copy(out_v, out_hbm.at[pl.ds(base3, TOK_W * DIM)])
    pltpu.sync_copy(acc_v, part_hbm.at[wid])


@functools.cache
def _sc_lookup():
    return pl.kernel(
        _sc_lookup_body,
        out_type=[
            jax.ShapeDtypeStruct((NTOK * DIM,), jnp.float32),
            jax.ShapeDtypeStruct((NW, LANES), jnp.float32),
        ],
        mesh=plsc.VectorSubcoreMesh(core_axis_name="c", subcore_axis_name="s",
                                    num_cores=NC, num_subcores=NS),
        compiler_params=pltpu.CompilerParams(needs_layout_passes=False),
        scratch_types=[
            pltpu.VMEM((VOCAB * DIM,), jnp.float32),
            pltpu.VMEM((TOK_W,), jnp.int32),
            pltpu.VMEM((TOK_W * DIM,), jnp.float32),
            pltpu.VMEM((TOK_W * DIM,), jnp.float32),
            pltpu.VMEM((LANES,), jnp.float32),
        ],
    )


def kernel(feats, embedding):
    B, L, D = feats.shape
    flat = feats.reshape(NTOK, DIM)
    flat_q = flat.astype(jnp.bfloat16).astype(jnp.float32)
    f2 = jnp.sum(flat ** 2, axis=1)[:, None]
    e2 = jnp.sum(embedding ** 2, axis=1)[None, :]
    idx = _tc_argmin(flat_q, embedding.T, f2, e2)
    qst_flat, partials = _sc_lookup()(
        embedding.reshape(-1), idx, flat.reshape(-1))
    quant_st = qst_flat.reshape(B, L, D)
    loss = jnp.sum(partials) * jnp.float32(1.25 / (NTOK * DIM))
    return (quant_st, idx.reshape(B, L), loss)


# speedup vs baseline: 2.0257x; 1.3788x over previous
import functools

import jax
import jax.numpy as jnp
from jax import lax
from jax.experimental import pallas as pl
from jax.experimental.pallas import tpu as pltpu
from jax.experimental.pallas import tpu_sc as plsc

VOCAB = 4096
DIM = 3
NTOK = 65536
TBLK = 1024
NC, NS, LANES = 2, 16, 16
NW = NC * NS
TOK_W = NTOK // NW
GROUPS = TOK_W // LANES


def _tc_argmin_body(ft_ref, em2_ref, f2_ref, e2_ref, idx_ref):
    ft = ft_ref[...]
    cvt = jax.lax.dot_general(
        em2_ref[...], ft, (((1,), (0,)), ((), ())),
        preferred_element_type=jnp.float32)
    dist = (f2_ref[...] + cvt) + e2_ref[...]
    m = jnp.min(dist, axis=0, keepdims=True)
    row = jax.lax.broadcasted_iota(jnp.int32, (VOCAB, 1), 0).astype(jnp.float32)
    sel = jnp.where(dist <= m, row, jnp.float32(VOCAB))
    idx_ref[...] = jnp.min(sel, axis=0, keepdims=True).astype(jnp.int32)


def _tc_argmin(ft_q, em2, f2, e2):
    return pl.pallas_call(
        _tc_argmin_body,
        grid=(NTOK // TBLK,),
        in_specs=[
            pl.BlockSpec((DIM, TBLK), lambda i: (0, i)),
            pl.BlockSpec((VOCAB, DIM), lambda i: (0, 0)),
            pl.BlockSpec((1, TBLK), lambda i: (0, i)),
            pl.BlockSpec((VOCAB, 1), lambda i: (0, 0)),
        ],
        out_specs=pl.BlockSpec((1, TBLK), lambda i: (0, i)),
        out_shape=jax.ShapeDtypeStruct((1, NTOK), jnp.int32),
    )(ft_q, em2, f2, e2)


def _sc_lookup_body(embt_hbm, idx_hbm, feats_hbm, out_hbm, part_hbm,
                    emb_v, idx_v, f0_v, f1_v, f2_v, o0_v, o1_v, o2_v, acc_v):
    wid = lax.axis_index("s") * NC + lax.axis_index("c")
    base = wid * TOK_W
    f_refs = (f0_v, f1_v, f2_v)
    o_refs = (o0_v, o1_v, o2_v)
    pltpu.sync_copy(embt_hbm, emb_v)
    pltpu.sync_copy(idx_hbm.at[pl.ds(base, TOK_W)], idx_v)
    for d in range(DIM):
        pltpu.sync_copy(feats_hbm.at[pl.ds(d * NTOK + base, TOK_W)], f_refs[d])

    def body(i, acc):
        iv = idx_v[pl.ds(i * LANES, LANES)]
        for d in range(DIM):
            fv = f_refs[d][pl.ds(i * LANES, LANES)]
            ev = plsc.load_gather(emb_v, [iv + d * VOCAB])
            dd = ev - fv
            o_refs[d][pl.ds(i * LANES, LANES)] = fv + dd
            acc = acc + dd * dd
        return acc

    acc = lax.fori_loop(0, GROUPS, body, jnp.zeros((LANES,), jnp.float32))
    acc_v[...] = acc
    for d in range(DIM):
        pltpu.sync_copy(o_refs[d], out_hbm.at[pl.ds(d * NTOK + base, TOK_W)])
    pltpu.sync_copy(acc_v, part_hbm.at[wid])


@functools.cache
def _sc_lookup():
    return pl.kernel(
        _sc_lookup_body,
        out_type=[
            jax.ShapeDtypeStruct((NTOK * DIM,), jnp.float32),
            jax.ShapeDtypeStruct((NW, LANES), jnp.float32),
        ],
        mesh=plsc.VectorSubcoreMesh(core_axis_name="c", subcore_axis_name="s",
                                    num_cores=NC, num_subcores=NS),
        compiler_params=pltpu.CompilerParams(needs_layout_passes=False),
        scratch_types=[
            pltpu.VMEM((VOCAB * DIM,), jnp.float32),
            pltpu.VMEM((TOK_W,), jnp.int32),
            pltpu.VMEM((TOK_W,), jnp.float32),
            pltpu.VMEM((TOK_W,), jnp.float32),
            pltpu.VMEM((TOK_W,), jnp.float32),
            pltpu.VMEM((TOK_W,), jnp.float32),
            pltpu.VMEM((TOK_W,), jnp.float32),
            pltpu.VMEM((TOK_W,), jnp.float32),
            pltpu.VMEM((LANES,), jnp.float32),
        ],
    )


def kernel(feats, embedding):
    B, L, D = feats.shape
    flat = feats.reshape(NTOK, DIM)
    ft = jnp.transpose(feats, (2, 0, 1)).reshape(DIM, NTOK)
    ft_q = ft.astype(jnp.bfloat16).astype(jnp.float32)
    f2 = jnp.sum(flat ** 2, axis=1)[None, :]
    e2 = jnp.sum(embedding ** 2, axis=1)[:, None]
    em2 = -2.0 * embedding
    idx2d = _tc_argmin(ft_q, em2, f2, e2)
    qst_t, partials = _sc_lookup()(
        jnp.transpose(embedding).reshape(-1),
        idx2d.reshape(-1),
        ft.reshape(-1))
    quant_st = jnp.transpose(qst_t.reshape(DIM, B, L), (1, 2, 0))
    loss = jnp.sum(partials) * jnp.float32(1.25 / (NTOK * DIM))
    return (quant_st, idx2d.reshape(B, L), loss)


# speedup vs baseline: 2.0862x; 1.0298x over previous
import functools

import jax
import jax.numpy as jnp
from jax import lax
from jax.experimental import pallas as pl
from jax.experimental.pallas import tpu as pltpu
from jax.experimental.pallas import tpu_sc as plsc

VOCAB = 4096
DIM = 3
NTOK = 65536
TBLK = 2048
NC, NS, LANES = 2, 16, 16
NW = NC * NS
TOK_W = NTOK // NW
GROUPS = TOK_W // LANES


def _tc_argmin_body(ft_ref, em2_ref, f2_ref, e2_ref, idx_ref):
    ft = ft_ref[...]
    cvt = jax.lax.dot_general(
        em2_ref[...], ft, (((1,), (0,)), ((), ())),
        preferred_element_type=jnp.float32)
    dist = (f2_ref[...] + cvt) + e2_ref[...]
    m = jnp.min(dist, axis=0, keepdims=True)
    row = jax.lax.broadcasted_iota(jnp.int32, (VOCAB, 1), 0).astype(jnp.float32)
    sel = jnp.where(dist <= m, row, jnp.float32(VOCAB))
    idx_ref[...] = jnp.min(sel, axis=0, keepdims=True).astype(jnp.int32)


def _tc_argmin(ft_q, em2, f2, e2):
    return pl.pallas_call(
        _tc_argmin_body,
        grid=(NTOK // TBLK,),
        in_specs=[
            pl.BlockSpec((DIM, TBLK), lambda i: (0, i)),
            pl.BlockSpec((VOCAB, DIM), lambda i: (0, 0)),
            pl.BlockSpec((1, TBLK), lambda i: (0, i)),
            pl.BlockSpec((VOCAB, 1), lambda i: (0, 0)),
        ],
        out_specs=pl.BlockSpec((1, TBLK), lambda i: (0, i)),
        out_shape=jax.ShapeDtypeStruct((1, NTOK), jnp.int32),
    )(ft_q, em2, f2, e2)


def _sc_lookup_body(embt_hbm, idx_hbm, feats_hbm, out_hbm, part_hbm,
                    emb_v, idx_v, f0_v, f1_v, f2_v, o0_v, o1_v, o2_v, acc_v):
    wid = lax.axis_index("s") * NC + lax.axis_index("c")
    base = wid * TOK_W
    f_refs = (f0_v, f1_v, f2_v)
    o_refs = (o0_v, o1_v, o2_v)
    pltpu.sync_copy(embt_hbm, emb_v)
    pltpu.sync_copy(idx_hbm.at[pl.ds(base, TOK_W)], idx_v)
    for d in range(DIM):
        pltpu.sync_copy(feats_hbm.at[pl.ds(d * NTOK + base, TOK_W)], f_refs[d])

    def body(i, acc):
        iv = idx_v[pl.ds(i * LANES, LANES)]
        for d in range(DIM):
            fv = f_refs[d][pl.ds(i * LANES, LANES)]
            ev = plsc.load_gather(emb_v, [iv + d * VOCAB])
            dd = ev - fv
            o_refs[d][pl.ds(i * LANES, LANES)] = fv + dd
            acc = acc + dd * dd
        return acc

    acc = lax.fori_loop(0, GROUPS, body, jnp.zeros((LANES,), jnp.float32))
    acc_v[...] = acc
    for d in range(DIM):
        pltpu.sync_copy(o_refs[d], out_hbm.at[pl.ds(d * NTOK + base, TOK_W)])
    pltpu.sync_copy(acc_v, part_hbm.at[wid])


@functools.cache
def _sc_lookup():
    return pl.kernel(
        _sc_lookup_body,
        out_type=[
            jax.ShapeDtypeStruct((NTOK * DIM,), jnp.float32),
            jax.ShapeDtypeStruct((NW, LANES), jnp.float32),
        ],
        mesh=plsc.VectorSubcoreMesh(core_axis_name="c", subcore_axis_name="s",
                                    num_cores=NC, num_subcores=NS),
        compiler_params=pltpu.CompilerParams(needs_layout_passes=False),
        scratch_types=[
            pltpu.VMEM((VOCAB * DIM,), jnp.float32),
            pltpu.VMEM((TOK_W,), jnp.int32),
            pltpu.VMEM((TOK_W,), jnp.float32),
            pltpu.VMEM((TOK_W,), jnp.float32),
            pltpu.VMEM((TOK_W,), jnp.float32),
            pltpu.VMEM((TOK_W,), jnp.float32),
            pltpu.VMEM((TOK_W,), jnp.float32),
            pltpu.VMEM((TOK_W,), jnp.float32),
            pltpu.VMEM((LANES,), jnp.float32),
        ],
    )


def kernel(feats, embedding):
    B, L, D = feats.shape
    flat = feats.reshape(NTOK, DIM)
    ft = jnp.transpose(feats, (2, 0, 1)).reshape(DIM, NTOK)
    ft_q = ft.astype(jnp.bfloat16).astype(jnp.float32)
    f2 = jnp.sum(flat ** 2, axis=1)[None, :]
    e2 = jnp.sum(embedding ** 2, axis=1)[:, None]
    em2 = -2.0 * embedding
    idx2d = _tc_argmin(ft_q, em2, f2, e2)
    qst_t, partials = _sc_lookup()(
        jnp.transpose(embedding).reshape(-1),
        idx2d.reshape(-1),
        ft.reshape(-1))
    quant_st = jnp.transpose(qst_t.reshape(DIM, B, L), (1, 2, 0))
    loss = jnp.sum(partials) * jnp.float32(1.25 / (NTOK * DIM))
    return (quant_st, idx2d.reshape(B, L), loss)


# speedup vs baseline: 2.0897x; 1.0017x over previous
import functools

import jax
import jax.numpy as jnp
from jax import lax
from jax.experimental import pallas as pl
from jax.experimental.pallas import tpu as pltpu
from jax.experimental.pallas import tpu_sc as plsc

VOCAB = 4096
DIM = 3
NTOK = 65536
TBLK = 2048
NC, NS, LANES = 2, 16, 16
NW = NC * NS
TOK_W = NTOK // NW
GROUPS = TOK_W // LANES


def _tc_argmin_body(ft_ref, em2_ref, f2_ref, e2_ref, idx_ref):
    ft = ft_ref[...].astype(jnp.bfloat16).astype(jnp.float32)
    cvt = jax.lax.dot_general(
        em2_ref[...], ft, (((1,), (0,)), ((), ())),
        preferred_element_type=jnp.float32)
    dist = (f2_ref[...] + cvt) + e2_ref[...]
    m = jnp.min(dist, axis=0, keepdims=True)
    row = jax.lax.broadcasted_iota(jnp.int32, (VOCAB, 1), 0).astype(jnp.float32)
    sel = jnp.where(dist <= m, row, jnp.float32(VOCAB))
    idx_ref[...] = jnp.min(sel, axis=0, keepdims=True).astype(jnp.int32)


def _tc_argmin(ft_q, em2, f2, e2):
    return pl.pallas_call(
        _tc_argmin_body,
        grid=(NTOK // TBLK,),
        in_specs=[
            pl.BlockSpec((DIM, TBLK), lambda i: (0, i)),
            pl.BlockSpec((VOCAB, DIM), lambda i: (0, 0)),
            pl.BlockSpec((1, TBLK), lambda i: (0, i)),
            pl.BlockSpec((VOCAB, 1), lambda i: (0, 0)),
        ],
        out_specs=pl.BlockSpec((1, TBLK), lambda i: (0, i)),
        out_shape=jax.ShapeDtypeStruct((1, NTOK), jnp.int32),
    )(ft_q, em2, f2, e2)


def _sc_lookup_body(embt_hbm, idx_hbm, feats_hbm, out_hbm, part_hbm,
                    emb_v, idx_v, f0_v, f1_v, f2_v, o0_v, o1_v, o2_v, acc_v):
    wid = lax.axis_index("s") * NC + lax.axis_index("c")
    base = wid * TOK_W
    f_refs = (f0_v, f1_v, f2_v)
    o_refs = (o0_v, o1_v, o2_v)
    pltpu.sync_copy(embt_hbm, emb_v)
    pltpu.sync_copy(idx_hbm.at[pl.ds(base, TOK_W)], idx_v)
    for d in range(DIM):
        pltpu.sync_copy(feats_hbm.at[pl.ds(d * NTOK + base, TOK_W)], f_refs[d])

    def body(i, acc):
        iv = idx_v[pl.ds(i * LANES, LANES)]
        for d in range(DIM):
            fv = f_refs[d][pl.ds(i * LANES, LANES)]
            ev = plsc.load_gather(emb_v, [iv + d * VOCAB])
            dd = ev - fv
            o_refs[d][pl.ds(i * LANES, LANES)] = fv + dd
            acc = acc + dd * dd
        return acc

    acc = lax.fori_loop(0, GROUPS, body, jnp.zeros((LANES,), jnp.float32))
    acc_v[...] = acc
    for d in range(DIM):
        pltpu.sync_copy(o_refs[d], out_hbm.at[pl.ds(d * NTOK + base, TOK_W)])
    pltpu.sync_copy(acc_v, part_hbm.at[wid])


@functools.cache
def _sc_lookup():
    return pl.kernel(
        _sc_lookup_body,
        out_type=[
            jax.ShapeDtypeStruct((NTOK * DIM,), jnp.float32),
            jax.ShapeDtypeStruct((NW, LANES), jnp.float32),
        ],
        mesh=plsc.VectorSubcoreMesh(core_axis_name="c", subcore_axis_name="s",
                                    num_cores=NC, num_subcores=NS),
        compiler_params=pltpu.CompilerParams(needs_layout_passes=False),
        scratch_types=[
            pltpu.VMEM((VOCAB * DIM,), jnp.float32),
            pltpu.VMEM((TOK_W,), jnp.int32),
            pltpu.VMEM((TOK_W,), jnp.float32),
            pltpu.VMEM((TOK_W,), jnp.float32),
            pltpu.VMEM((TOK_W,), jnp.float32),
            pltpu.VMEM((TOK_W,), jnp.float32),
            pltpu.VMEM((TOK_W,), jnp.float32),
            pltpu.VMEM((TOK_W,), jnp.float32),
            pltpu.VMEM((LANES,), jnp.float32),
        ],
    )


def kernel(feats, embedding):
    B, L, D = feats.shape
    flat = feats.reshape(NTOK, DIM)
    ft = jnp.transpose(feats, (2, 0, 1)).reshape(DIM, NTOK)
    f2 = jnp.sum(flat ** 2, axis=1)[None, :]
    e2 = jnp.sum(embedding ** 2, axis=1)[:, None]
    em2 = -2.0 * embedding
    idx2d = _tc_argmin(ft, em2, f2, e2)
    qst_t, partials = _sc_lookup()(
        jnp.transpose(embedding).reshape(-1),
        idx2d.reshape(-1),
        ft.reshape(-1))
    quant_st = jnp.transpose(qst_t.reshape(DIM, B, L), (1, 2, 0))
    loss = jnp.sum(partials) * jnp.float32(1.25 / (NTOK * DIM))
    return (quant_st, idx2d.reshape(B, L), loss)
